# Initial kernel scaffold; baseline (speedup 1.0000x reference)
import functools

import jax
import jax.numpy as jnp
from jax import lax
from jax.experimental import pallas as pl
from jax.experimental.pallas import tpu as pltpu
from jax.experimental.pallas import tpu_sc as plsc

N = 10000
E = 160000
F_IN = 256
HID = 256
F_OUT = 128

NS = 16
E_PAD = 163840
CHUNK = 128
NCHUNK = E_PAD // (NS * CHUNK)
EDGES_PER_W32 = E_PAD // 32
ACC_ROWS = 10240
DUMMY = N
ROWS_PER_TILE = N // NS
BLK = 2000

_f32 = jnp.float32


def _vsmesh():
    return plsc.VectorSubcoreMesh(core_axis_name="c", subcore_axis_name="s")


def _deg_call(dst32):
    @functools.partial(
        pl.kernel,
        out_type=jax.ShapeDtypeStruct((32, N), _f32),
        mesh=_vsmesh(),
        scratch_types=[
            pltpu.VMEM((EDGES_PER_W32,), jnp.int32),
            pltpu.VMEM((10016,), _f32),
        ],
    )
    def deg_kernel(dst_hbm, out_hbm, dstv, histv):
        c = lax.axis_index("c")
        s = lax.axis_index("s")
        w = c * NS + s
        pltpu.sync_copy(dst_hbm.at[w], dstv)
        zf = jnp.zeros((16,), _f32)
        onef = jnp.ones((16,), _f32)

        @pl.loop(0, 10016 // 16)
        def _(i):
            histv[pl.ds(i * 16, 16)] = zf

        @pl.loop(0, EDGES_PER_W32 // 16)
        def _(i):
            idx = dstv[pl.ds(i * 16, 16)]
            plsc.addupdate_scatter(histv, [idx], onef)

        pltpu.sync_copy(histv.at[pl.ds(0, N)], out_hbm.at[w])

    return deg_kernel(dst32)


def _agg_call(hs_lo, hs_hi, src_t, dst_t, f_half):
    @functools.partial(
        pl.kernel,
        out_type=(
            jax.ShapeDtypeStruct((N, f_half), _f32),
            jax.ShapeDtypeStruct((N, f_half), _f32),
        ),
        mesh=_vsmesh(),
        scratch_types=[
            pltpu.VMEM((NCHUNK, CHUNK), jnp.int32),
            pltpu.VMEM((NCHUNK, CHUNK), jnp.int32),
            pltpu.VMEM((CHUNK, f_half), _f32),
            pltpu.VMEM_SHARED((ACC_ROWS, f_half), _f32),
            pltpu.SemaphoreType.DMA,
        ],
    )
    def agg_kernel(lo_hbm, hi_hbm, src_hbm, dst_hbm, out_lo, out_hi,
                   srcv, dstv, rows, acc, sem):
        c = lax.axis_index("c")
        s = lax.axis_index("s")
        pltpu.sync_copy(src_hbm.at[s], srcv)
        pltpu.sync_copy(dst_hbm.at[s], dstv)

        zf = jnp.zeros((16,), _f32)

        @pl.loop(0, CHUNK)
        def _(r):
            @pl.loop(0, f_half // 16)
            def _(q):
                rows[r, pl.ds(q * 16, 16)] = zf

        @pl.loop(0, ACC_ROWS // NS // CHUNK)
        def _(k):
            pltpu.sync_copy(
                rows, acc.at[pl.ds(s * (ACC_ROWS // NS) + k * CHUNK, CHUNK)])

        plsc.subcore_barrier()

        def run(hs, out):
            @pl.loop(0, NCHUNK)
            def _(j):
                pltpu.async_copy(hs.at[srcv.at[j]], rows, sem).wait()
                pltpu.sync_copy(rows, acc.at[dstv.at[j]], add=True)

            plsc.subcore_barrier()
            pltpu.sync_copy(acc.at[pl.ds(s * ROWS_PER_TILE, ROWS_PER_TILE)],
                            out.at[pl.ds(s * ROWS_PER_TILE, ROWS_PER_TILE)])

        @pl.when(c == 0)
        def _():
            run(lo_hbm, out_lo)

        @pl.when(c == 1)
        def _():
            run(hi_hbm, out_hi)

    return agg_kernel(hs_lo, hs_hi, src_t, dst_t)


_DOT = functools.partial(
    lax.dot_general,
    precision=lax.Precision.HIGHEST,
    preferred_element_type=_f32,
)


def _mm_body(x_ref, w_ref, o_ref):
    o_ref[...] = _DOT(x_ref[...], w_ref[...], (((1,), (0,)), ((), ())))


def _mm_call(x, w):
    m, k = x.shape
    n = w.shape[1]
    return pl.pallas_call(
        _mm_body,
        grid=(m // BLK,),
        in_specs=[pl.BlockSpec((BLK, k), lambda i: (i, 0)),
                  pl.BlockSpec((k, n), lambda i: (0, 0))],
        out_specs=pl.BlockSpec((BLK, n), lambda i: (i, 0)),
        out_shape=jax.ShapeDtypeStruct((m, n), _f32),
    )(x, w)


def _scale1_body(p_ref, h_ref, lo_ref, hi_ref, dv_ref):
    ones = jnp.ones((32, 1), _f32)
    deg = _DOT(p_ref[...], ones, (((0,), (0,)), ((), ()))) + 1.0
    dinv = lax.rsqrt(deg)
    hs = h_ref[...] * dinv
    lo_ref[...] = hs[:, :HID // 2]
    hi_ref[...] = hs[:, HID // 2:]
    dv_ref[...] = dinv


def _scale1_call(partials, h1):
    return pl.pallas_call(
        _scale1_body,
        grid=(N // BLK,),
        in_specs=[pl.BlockSpec((32, BLK), lambda i: (0, i)),
                  pl.BlockSpec((BLK, HID), lambda i: (i, 0))],
        out_specs=[pl.BlockSpec((BLK, HID // 2), lambda i: (i, 0)),
                   pl.BlockSpec((BLK, HID // 2), lambda i: (i, 0)),
                   pl.BlockSpec((BLK, 1), lambda i: (i, 0))],
        out_shape=[jax.ShapeDtypeStruct((N, HID // 2), _f32),
                   jax.ShapeDtypeStruct((N, HID // 2), _f32),
                   jax.ShapeDtypeStruct((N, 1), _f32)],
    )(partials, h1)


def _layer_body(lo_ref, hi_ref, h1_ref, dv_ref, b1_ref, w2_ref,
                h2_ref, lo2_ref, hi2_ref):
    s1 = jnp.concatenate([lo_ref[...], hi_ref[...]], axis=1)
    dinv = dv_ref[...]
    out1 = dinv * s1 + (dinv * dinv) * h1_ref[...] + b1_ref[...]
    h = jnp.maximum(out1, 0.0)
    h2 = _DOT(h, w2_ref[...], (((1,), (0,)), ((), ())))
    h2_ref[...] = h2
    hs2 = dinv * h2
    lo2_ref[...] = hs2[:, :F_OUT // 2]
    hi2_ref[...] = hs2[:, F_OUT // 2:]


def _layer_call(s1_lo, s1_hi, h1, dinv, b1, w2):
    return pl.pallas_call(
        _layer_body,
        grid=(N // BLK,),
        in_specs=[pl.BlockSpec((BLK, HID // 2), lambda i: (i, 0)),
                  pl.BlockSpec((BLK, HID // 2), lambda i: (i, 0)),
                  pl.BlockSpec((BLK, HID), lambda i: (i, 0)),
                  pl.BlockSpec((BLK, 1), lambda i: (i, 0)),
                  pl.BlockSpec((1, HID), lambda i: (0, 0)),
                  pl.BlockSpec((HID, F_OUT), lambda i: (0, 0))],
        out_specs=[pl.BlockSpec((BLK, F_OUT), lambda i: (i, 0)),
                   pl.BlockSpec((BLK, F_OUT // 2), lambda i: (i, 0)),
                   pl.BlockSpec((BLK, F_OUT // 2), lambda i: (i, 0))],
        out_shape=[jax.ShapeDtypeStruct((N, F_OUT), _f32),
                   jax.ShapeDtypeStruct((N, F_OUT // 2), _f32),
                   jax.ShapeDtypeStruct((N, F_OUT // 2), _f32)],
    )(s1_lo, s1_hi, h1, dinv, b1, w2)


def _final_body(lo_ref, hi_ref, h2_ref, dv_ref, b2_ref, o_ref):
    s2 = jnp.concatenate([lo_ref[...], hi_ref[...]], axis=1)
    dinv = dv_ref[...]
    o_ref[...] = dinv * s2 + (dinv * dinv) * h2_ref[...] + b2_ref[...]


def _final_call(s2_lo, s2_hi, h2, dinv, b2):
    return pl.pallas_call(
        _final_body,
        grid=(N // BLK,),
        in_specs=[pl.BlockSpec((BLK, F_OUT // 2), lambda i: (i, 0)),
                  pl.BlockSpec((BLK, F_OUT // 2), lambda i: (i, 0)),
                  pl.BlockSpec((BLK, F_OUT), lambda i: (i, 0)),
                  pl.BlockSpec((BLK, 1), lambda i: (i, 0)),
                  pl.BlockSpec((1, F_OUT), lambda i: (0, 0))],
        out_specs=pl.BlockSpec((BLK, F_OUT), lambda i: (i, 0)),
        out_shape=jax.ShapeDtypeStruct((N, F_OUT), _f32),
    )(s2_lo, s2_hi, h2, dinv, b2)


def kernel(x, edge_index, W1, b1, W2, b2):
    src = edge_index[0]
    dst = edge_index[1]
    pad = E_PAD - E
    src_p = jnp.concatenate([src, jnp.zeros((pad,), jnp.int32)])
    dst_p = jnp.concatenate([dst, jnp.full((pad,), DUMMY, jnp.int32)])
    src_t = src_p.reshape(NS, NCHUNK, CHUNK)
    dst_t = dst_p.reshape(NS, NCHUNK, CHUNK)
    dst32 = dst_p.reshape(32, EDGES_PER_W32)

    partials = _deg_call(dst32)
    h1 = _mm_call(x, W1)
    hs1_lo, hs1_hi, dinv = _scale1_call(partials, h1)
    s1_lo, s1_hi = _agg_call(hs1_lo, hs1_hi, src_t, dst_t, HID // 2)
    h2, hs2_lo, hs2_hi = _layer_call(s1_lo, s1_hi, h1, dinv,
                                     b1.reshape(1, HID), W2)
    s2_lo, s2_hi = _agg_call(hs2_lo, hs2_hi, src_t, dst_t, F_OUT // 2)
    return _final_call(s2_lo, s2_hi, h2, dinv, b2.reshape(1, F_OUT))


# speedup vs baseline: 8.0794x; 8.0794x over previous
import dataclasses
import functools

import jax
import jax.numpy as jnp
from jax import lax
from jax.experimental import pallas as pl
from jax.experimental.pallas import tpu as pltpu
from jax.experimental.pallas import tpu_sc as plsc

N = 10000
E = 160000
F_IN = 256
HID = 256
F_OUT = 128

NS = 16
E_PAD = 163840
CHUNK = 128
NCHUNK = E_PAD // (NS * CHUNK)
EDGES_PER_W32 = E_PAD // 32
ACC_ROWS = 10240
DUMMY = N
ROWS_PER_TILE = N // NS
BLK = 2000

_f32 = jnp.float32


def _vsmesh():
    return plsc.VectorSubcoreMesh(core_axis_name="c", subcore_axis_name="s")


def _sc_compiler_params(layout_passes=True):
    return pltpu.CompilerParams(
        use_tc_tiling_on_sc=False,
        needs_layout_passes=layout_passes,
    )


def _deg_call(dst32):
    @functools.partial(
        pl.kernel,
        out_type=jax.ShapeDtypeStruct((32, N), _f32),
        mesh=_vsmesh(),
        scratch_types=[
            pltpu.VMEM((EDGES_PER_W32,), jnp.int32),
            pltpu.VMEM((10016,), _f32),
        ],
        compiler_params=_sc_compiler_params(layout_passes=False),
    )
    def deg_kernel(dst_hbm, out_hbm, dstv, histv):
        c = lax.axis_index("c")
        s = lax.axis_index("s")
        w = c * NS + s
        pltpu.sync_copy(dst_hbm.at[w], dstv)
        zf = jnp.zeros((16,), _f32)
        onef = jnp.ones((16,), _f32)

        @pl.loop(0, 10016 // 16)
        def _(i):
            histv[pl.ds(i * 16, 16)] = zf

        @pl.loop(0, EDGES_PER_W32 // 16)
        def _(i):
            idx = dstv[pl.ds(i * 16, 16)]
            plsc.addupdate_scatter(histv, [idx], onef)

        pltpu.sync_copy(histv.at[pl.ds(0, N)], out_hbm.at[w])

    return deg_kernel(dst32)


def _agg_call(hs_lo, hs_hi, src_t, dst_t, f_half):
    @functools.partial(
        pl.kernel,
        out_type=(
            jax.ShapeDtypeStruct((N, f_half), _f32),
            jax.ShapeDtypeStruct((N, f_half), _f32),
        ),
        mesh=_vsmesh(),
        scratch_types=[
            pltpu.VMEM((NCHUNK, CHUNK), jnp.int32),
            pltpu.VMEM((NCHUNK, CHUNK), jnp.int32),
            pltpu.VMEM((CHUNK, f_half), _f32),
            pltpu.VMEM_SHARED((ACC_ROWS, f_half), _f32),
            pltpu.SemaphoreType.DMA,
        ],
        compiler_params=_sc_compiler_params(),
    )
    def agg_kernel(lo_hbm, hi_hbm, src_hbm, dst_hbm, out_lo, out_hi,
                   srcv, dstv, rows, acc, sem):
        c = lax.axis_index("c")
        s = lax.axis_index("s")
        pltpu.sync_copy(src_hbm.at[s], srcv)
        pltpu.sync_copy(dst_hbm.at[s], dstv)

        zf = jnp.zeros((16,), _f32)

        @pl.loop(0, CHUNK)
        def _(r):
            @pl.loop(0, f_half // 16)
            def _(q):
                rows[r, pl.ds(q * 16, 16)] = zf

        @pl.loop(0, ACC_ROWS // NS // CHUNK)
        def _(k):
            pltpu.sync_copy(
                rows, acc.at[pl.ds(s * (ACC_ROWS // NS) + k * CHUNK, CHUNK)])

        plsc.subcore_barrier()

        def run(hs, out):
            @pl.loop(0, NCHUNK)
            def _(j):
                pltpu.async_copy(hs.at[srcv.at[j]], rows, sem).wait()
                pltpu.sync_copy(rows, acc.at[dstv.at[j]], add=True)

            plsc.subcore_barrier()
            pltpu.sync_copy(acc.at[pl.ds(s * ROWS_PER_TILE, ROWS_PER_TILE)],
                            out.at[pl.ds(s * ROWS_PER_TILE, ROWS_PER_TILE)])

        @pl.when(c == 0)
        def _():
            run(lo_hbm, out_lo)

        @pl.when(c == 1)
        def _():
            run(hi_hbm, out_hi)

    return agg_kernel(hs_lo, hs_hi, src_t, dst_t)


_DOT = functools.partial(
    lax.dot_general,
    precision=lax.Precision.HIGHEST,
    preferred_element_type=_f32,
)


def _mm_body(x_ref, w_ref, o_ref):
    o_ref[...] = _DOT(x_ref[...], w_ref[...], (((1,), (0,)), ((), ())))


def _mm_call(x, w):
    m, k = x.shape
    n = w.shape[1]
    return pl.pallas_call(
        _mm_body,
        grid=(m // BLK,),
        in_specs=[pl.BlockSpec((BLK, k), lambda i: (i, 0)),
                  pl.BlockSpec((k, n), lambda i: (0, 0))],
        out_specs=pl.BlockSpec((BLK, n), lambda i: (i, 0)),
        out_shape=jax.ShapeDtypeStruct((m, n), _f32),
    )(x, w)


def _dinv_body(p_ref, dv_ref):
    ones = jnp.ones((32, 1), _f32)
    deg = _DOT(p_ref[...], ones, (((0,), (0,)), ((), ()))) + 1.0
    dv_ref[...] = lax.rsqrt(deg)


def _dinv_call(partials):
    return pl.pallas_call(
        _dinv_body,
        in_specs=[pl.BlockSpec((32, N), lambda: (0, 0))],
        out_specs=pl.BlockSpec((N, 1), lambda: (0, 0)),
        out_shape=jax.ShapeDtypeStruct((N, 1), _f32),
    )(partials)


def _scale1_body(dv_ref, h_ref, lo_ref, hi_ref):
    hs = h_ref[...] * dv_ref[...]
    lo_ref[...] = hs[:, :HID // 2]
    hi_ref[...] = hs[:, HID // 2:]


def _scale1_call(dinv, h1):
    return pl.pallas_call(
        _scale1_body,
        grid=(N // BLK,),
        in_specs=[pl.BlockSpec((BLK, 1), lambda i: (i, 0)),
                  pl.BlockSpec((BLK, HID), lambda i: (i, 0))],
        out_specs=[pl.BlockSpec((BLK, HID // 2), lambda i: (i, 0)),
                   pl.BlockSpec((BLK, HID // 2), lambda i: (i, 0))],
        out_shape=[jax.ShapeDtypeStruct((N, HID // 2), _f32),
                   jax.ShapeDtypeStruct((N, HID // 2), _f32)],
    )(dinv, h1)


def _layer_body(lo_ref, hi_ref, h1_ref, dv_ref, b1_ref, w2_ref,
                h2_ref, lo2_ref, hi2_ref):
    s1 = jnp.concatenate([lo_ref[...], hi_ref[...]], axis=1)
    dinv = dv_ref[...]
    out1 = dinv * s1 + (dinv * dinv) * h1_ref[...] + b1_ref[...]
    h = jnp.maximum(out1, 0.0)
    h2 = _DOT(h, w2_ref[...], (((1,), (0,)), ((), ())))
    h2_ref[...] = h2
    hs2 = dinv * h2
    lo2_ref[...] = hs2[:, :F_OUT // 2]
    hi2_ref[...] = hs2[:, F_OUT // 2:]


def _layer_call(s1_lo, s1_hi, h1, dinv, b1, w2):
    return pl.pallas_call(
        _layer_body,
        grid=(N // BLK,),
        in_specs=[pl.BlockSpec((BLK, HID // 2), lambda i: (i, 0)),
                  pl.BlockSpec((BLK, HID // 2), lambda i: (i, 0)),
                  pl.BlockSpec((BLK, HID), lambda i: (i, 0)),
                  pl.BlockSpec((BLK, 1), lambda i: (i, 0)),
                  pl.BlockSpec((1, HID), lambda i: (0, 0)),
                  pl.BlockSpec((HID, F_OUT), lambda i: (0, 0))],
        out_specs=[pl.BlockSpec((BLK, F_OUT), lambda i: (i, 0)),
                   pl.BlockSpec((BLK, F_OUT // 2), lambda i: (i, 0)),
                   pl.BlockSpec((BLK, F_OUT // 2), lambda i: (i, 0))],
        out_shape=[jax.ShapeDtypeStruct((N, F_OUT), _f32),
                   jax.ShapeDtypeStruct((N, F_OUT // 2), _f32),
                   jax.ShapeDtypeStruct((N, F_OUT // 2), _f32)],
    )(s1_lo, s1_hi, h1, dinv, b1, w2)


def _final_body(lo_ref, hi_ref, h2_ref, dv_ref, b2_ref, o_ref):
    s2 = jnp.concatenate([lo_ref[...], hi_ref[...]], axis=1)
    dinv = dv_ref[...]
    o_ref[...] = dinv * s2 + (dinv * dinv) * h2_ref[...] + b2_ref[...]


def _final_call(s2_lo, s2_hi, h2, dinv, b2):
    return pl.pallas_call(
        _final_body,
        grid=(N // BLK,),
        in_specs=[pl.BlockSpec((BLK, F_OUT // 2), lambda i: (i, 0)),
                  pl.BlockSpec((BLK, F_OUT // 2), lambda i: (i, 0)),
                  pl.BlockSpec((BLK, F_OUT), lambda i: (i, 0)),
                  pl.BlockSpec((BLK, 1), lambda i: (i, 0)),
                  pl.BlockSpec((1, F_OUT), lambda i: (0, 0))],
        out_specs=pl.BlockSpec((BLK, F_OUT), lambda i: (i, 0)),
        out_shape=jax.ShapeDtypeStruct((N, F_OUT), _f32),
    )(s2_lo, s2_hi, h2, dinv, b2)


def kernel(x, edge_index, W1, b1, W2, b2):
    src = edge_index[0]
    dst = edge_index[1]
    pad = E_PAD - E
    src_p = jnp.concatenate([src, jnp.zeros((pad,), jnp.int32)])
    dst_p = jnp.concatenate([dst, jnp.full((pad,), DUMMY, jnp.int32)])
    src_t = src_p.reshape(NS, NCHUNK, CHUNK)
    dst_t = dst_p.reshape(NS, NCHUNK, CHUNK)
    dst32 = dst_p.reshape(32, EDGES_PER_W32)

    partials = _deg_call(dst32)
    h1 = _mm_call(x, W1)
    dinv = _dinv_call(partials)
    hs1_lo, hs1_hi = _scale1_call(dinv, h1)
    s1_lo, s1_hi = _agg_call(hs1_lo, hs1_hi, src_t, dst_t, HID // 2)
    h2, hs2_lo, hs2_hi = _layer_call(s1_lo, s1_hi, h1, dinv,
                                     b1.reshape(1, HID), W2)
    s2_lo, s2_hi = _agg_call(hs2_lo, hs2_hi, src_t, dst_t, F_OUT // 2)
    return _final_call(s2_lo, s2_hi, h2, dinv, b2.reshape(1, F_OUT))


# speedup vs baseline: 11.5688x; 1.4319x over previous
import dataclasses
import functools

import jax
import jax.numpy as jnp
from jax import lax
from jax.experimental import pallas as pl
from jax.experimental.pallas import tpu as pltpu
from jax.experimental.pallas import tpu_sc as plsc

N = 10000
E = 160000
F_IN = 256
HID = 256
F_OUT = 128

NS = 16
E_PAD = 161280
CHUNK = 112
NCHUNK = E_PAD // (NS * CHUNK)
EDGES_PER_W32 = E_PAD // 32
ACC_ROWS = 10016
DUMMY = N
ROWS_PER_TILE = N // NS
BLK = 2000

_f32 = jnp.float32


def _vsmesh():
    return plsc.VectorSubcoreMesh(core_axis_name="c", subcore_axis_name="s")


def _sc_compiler_params(layout_passes=True):
    return pltpu.CompilerParams(
        use_tc_tiling_on_sc=False,
        needs_layout_passes=layout_passes,
        internal_scratch_in_bytes=0,
    )


def _deg_call(dst32):
    @functools.partial(
        pl.kernel,
        out_type=jax.ShapeDtypeStruct((32, N), _f32),
        mesh=_vsmesh(),
        scratch_types=[
            pltpu.VMEM((EDGES_PER_W32,), jnp.int32),
            pltpu.VMEM((10016,), _f32),
        ],
        compiler_params=_sc_compiler_params(layout_passes=False),
    )
    def deg_kernel(dst_hbm, out_hbm, dstv, histv):
        c = lax.axis_index("c")
        s = lax.axis_index("s")
        w = c * NS + s
        pltpu.sync_copy(dst_hbm.at[w], dstv)
        zf = jnp.zeros((16,), _f32)
        onef = jnp.ones((16,), _f32)

        @pl.loop(0, 10016 // 16)
        def _(i):
            histv[pl.ds(i * 16, 16)] = zf

        @pl.loop(0, EDGES_PER_W32 // 16)
        def _(i):
            idx = dstv[pl.ds(i * 16, 16)]
            plsc.addupdate_scatter(histv, [idx], onef)

        pltpu.sync_copy(histv.at[pl.ds(0, N)], out_hbm.at[w])

    return deg_kernel(dst32)


def _agg_call(hs2, src2_t, dst_t, f_half):
    @functools.partial(
        pl.kernel,
        out_type=jax.ShapeDtypeStruct((2 * NS, ROWS_PER_TILE, f_half), _f32),
        mesh=_vsmesh(),
        scratch_types=[
            pltpu.VMEM((NCHUNK, CHUNK), jnp.int32),
            pltpu.VMEM((NCHUNK, CHUNK), jnp.int32),
            pltpu.VMEM((CHUNK, f_half), _f32),
            pltpu.VMEM((CHUNK, f_half), _f32),
            pltpu.VMEM_SHARED((ACC_ROWS, f_half), _f32),
            pltpu.SemaphoreType.DMA,
            pltpu.SemaphoreType.DMA,
        ],
        compiler_params=_sc_compiler_params(),
    )
    def agg_kernel(hs_hbm, src_hbm, dst_hbm, out_hbm,
                   srcv, dstv, rows, rows1, acc, sem, sem1):
        c = lax.axis_index("c")
        s = lax.axis_index("s")
        w = c * NS + s
        pltpu.sync_copy(src_hbm.at[w], srcv)
        pltpu.sync_copy(dst_hbm.at[s], dstv)

        zf = jnp.zeros((16,), _f32)

        @pl.loop(0, CHUNK)
        def _(r):
            @pl.loop(0, f_half // 16)
            def _(q):
                rows[r, pl.ds(q * 16, 16)] = zf

        base = s * (ACC_ROWS // NS)

        @pl.loop(0, 5)
        def _(k):
            pltpu.sync_copy(rows, acc.at[pl.ds(base + k * CHUNK, CHUNK)])

        pltpu.sync_copy(rows.at[pl.ds(0, 66)],
                        acc.at[pl.ds(base + 5 * CHUNK, 66)])

        plsc.subcore_barrier()

        pltpu.async_copy(hs_hbm.at[srcv.at[0]], rows, sem)

        @pl.loop(0, NCHUNK // 2)
        def _(i):
            j0 = 2 * i
            j1 = j0 + 1
            pltpu.make_async_copy(hs_hbm.at[srcv.at[j0]], rows, sem).wait()
            pltpu.async_copy(hs_hbm.at[srcv.at[j1]], rows1, sem1)
            pltpu.sync_copy(rows, acc.at[dstv.at[j0]], add=True)
            pltpu.make_async_copy(hs_hbm.at[srcv.at[j1]], rows1, sem1).wait()

            @pl.when(j1 + 1 < NCHUNK)
            def _():
                pltpu.async_copy(hs_hbm.at[srcv.at[j1 + 1]], rows, sem)

            pltpu.sync_copy(rows1, acc.at[dstv.at[j1]], add=True)

        plsc.subcore_barrier()
        pltpu.sync_copy(acc.at[pl.ds(s * ROWS_PER_TILE, ROWS_PER_TILE)],
                        out_hbm.at[w])

    return agg_kernel(hs2, src2_t, dst_t)


_DOT = functools.partial(
    lax.dot_general,
    precision=lax.Precision.HIGHEST,
    preferred_element_type=_f32,
)


def _mm_body(x_ref, w_ref, o_ref):
    o_ref[...] = _DOT(x_ref[...], w_ref[...], (((1,), (0,)), ((), ())))


def _mm_call(x, w):
    m, k = x.shape
    n = w.shape[1]
    return pl.pallas_call(
        _mm_body,
        grid=(m // BLK,),
        in_specs=[pl.BlockSpec((BLK, k), lambda i: (i, 0)),
                  pl.BlockSpec((k, n), lambda i: (0, 0))],
        out_specs=pl.BlockSpec((BLK, n), lambda i: (i, 0)),
        out_shape=jax.ShapeDtypeStruct((m, n), _f32),
    )(x, w)


def _dinv_body(p_ref, dv_ref):
    ones = jnp.ones((32, 1), _f32)
    deg = _DOT(p_ref[...], ones, (((0,), (0,)), ((), ()))) + 1.0
    dv_ref[...] = lax.rsqrt(deg)


def _dinv_call(partials):
    return pl.pallas_call(
        _dinv_body,
        in_specs=[pl.BlockSpec((32, N), lambda: (0, 0))],
        out_specs=pl.BlockSpec((N, 1), lambda: (0, 0)),
        out_shape=jax.ShapeDtypeStruct((N, 1), _f32),
    )(partials)


def _scale1_body(dv_ref, h_ref, o_ref):
    hs = h_ref[...] * dv_ref[...]
    o_ref[0] = hs[:, :HID // 2]
    o_ref[1] = hs[:, HID // 2:]


def _scale1_call(dinv, h1):
    return pl.pallas_call(
        _scale1_body,
        grid=(N // BLK,),
        in_specs=[pl.BlockSpec((BLK, 1), lambda i: (i, 0)),
                  pl.BlockSpec((BLK, HID), lambda i: (i, 0))],
        out_specs=pl.BlockSpec((2, BLK, HID // 2), lambda i: (0, i, 0)),
        out_shape=jax.ShapeDtypeStruct((2, N, HID // 2), _f32),
    )(dinv, h1)


def _layer_body(lo_ref, hi_ref, h1_ref, dv_ref, b1_ref, w2_ref,
                h2_ref, o2_ref):
    s1 = jnp.concatenate([lo_ref[...], hi_ref[...]], axis=1)
    dinv = dv_ref[...]
    out1 = dinv * s1 + (dinv * dinv) * h1_ref[...] + b1_ref[...]
    h = jnp.maximum(out1, 0.0)
    h2 = _DOT(h, w2_ref[...], (((1,), (0,)), ((), ())))
    h2_ref[...] = h2
    hs2 = dinv * h2
    o2_ref[0] = hs2[:, :F_OUT // 2]
    o2_ref[1] = hs2[:, F_OUT // 2:]


def _layer_call(s1_lo, s1_hi, h1, dinv, b1, w2):
    return pl.pallas_call(
        _layer_body,
        grid=(N // BLK,),
        in_specs=[pl.BlockSpec((BLK, HID // 2), lambda i: (i, 0)),
                  pl.BlockSpec((BLK, HID // 2), lambda i: (i, 0)),
                  pl.BlockSpec((BLK, HID), lambda i: (i, 0)),
                  pl.BlockSpec((BLK, 1), lambda i: (i, 0)),
                  pl.BlockSpec((1, HID), lambda i: (0, 0)),
                  pl.BlockSpec((HID, F_OUT), lambda i: (0, 0))],
        out_specs=[pl.BlockSpec((BLK, F_OUT), lambda i: (i, 0)),
                   pl.BlockSpec((2, BLK, F_OUT // 2), lambda i: (0, i, 0))],
        out_shape=[jax.ShapeDtypeStruct((N, F_OUT), _f32),
                   jax.ShapeDtypeStruct((2, N, F_OUT // 2), _f32)],
    )(s1_lo, s1_hi, h1, dinv, b1, w2)


def _final_body(lo_ref, hi_ref, h2_ref, dv_ref, b2_ref, o_ref):
    s2 = jnp.concatenate([lo_ref[...], hi_ref[...]], axis=1)
    dinv = dv_ref[...]
    o_ref[...] = dinv * s2 + (dinv * dinv) * h2_ref[...] + b2_ref[...]


def _final_call(s2_lo, s2_hi, h2, dinv, b2):
    return pl.pallas_call(
        _final_body,
        grid=(N // BLK,),
        in_specs=[pl.BlockSpec((BLK, F_OUT // 2), lambda i: (i, 0)),
                  pl.BlockSpec((BLK, F_OUT // 2), lambda i: (i, 0)),
                  pl.BlockSpec((BLK, F_OUT), lambda i: (i, 0)),
                  pl.BlockSpec((BLK, 1), lambda i: (i, 0)),
                  pl.BlockSpec((1, F_OUT), lambda i: (0, 0))],
        out_specs=pl.BlockSpec((BLK, F_OUT), lambda i: (i, 0)),
        out_shape=jax.ShapeDtypeStruct((N, F_OUT), _f32),
    )(s2_lo, s2_hi, h2, dinv, b2)


def kernel(x, edge_index, W1, b1, W2, b2):
    src = edge_index[0]
    dst = edge_index[1]
    pad = E_PAD - E
    src_p = jnp.concatenate([src, jnp.zeros((pad,), jnp.int32)])
    dst_p = jnp.concatenate([dst, jnp.full((pad,), DUMMY, jnp.int32)])
    src2_t = jnp.stack([src_p, src_p + N]).reshape(2 * NS, NCHUNK, CHUNK)
    dst_t = dst_p.reshape(NS, NCHUNK, CHUNK)
    dst32 = dst_p.reshape(32, EDGES_PER_W32)

    partials = _deg_call(dst32)
    h1 = _mm_call(x, W1)
    dinv = _dinv_call(partials)
    hs1 = _scale1_call(dinv, h1).reshape(2 * N, HID // 2)
    s1 = _agg_call(hs1, src2_t, dst_t, HID // 2).reshape(2, N, HID // 2)
    h2, hs2 = _layer_call(s1[0], s1[1], h1, dinv,
                          b1.reshape(1, HID), W2)
    s2 = _agg_call(hs2.reshape(2 * N, F_OUT // 2), src2_t, dst_t,
                   F_OUT // 2).reshape(2, N, F_OUT // 2)
    return _final_call(s2[0], s2[1], h2, dinv, b2.reshape(1, F_OUT))


# speedup vs baseline: 13.9303x; 1.2041x over previous
import dataclasses
import functools

import jax
import jax.numpy as jnp
from jax import lax
from jax.experimental import pallas as pl
from jax.experimental.pallas import tpu as pltpu
from jax.experimental.pallas import tpu_sc as plsc

N = 10000
E = 160000
F_IN = 256
HID = 256
F_OUT = 128

NS = 16
E_PAD = 161280
CHUNK = 80
NCHUNK = E_PAD // (NS * CHUNK)
EDGES_PER_W32 = E_PAD // 32
ACC_ROWS = 10016
DUMMY = N
ROWS_PER_TILE = N // NS
BLK = 2000

_f32 = jnp.float32


def _vsmesh():
    return plsc.VectorSubcoreMesh(core_axis_name="c", subcore_axis_name="s")


def _sc_compiler_params(layout_passes=True):
    return pltpu.CompilerParams(
        use_tc_tiling_on_sc=False,
        needs_layout_passes=layout_passes,
        internal_scratch_in_bytes=0,
    )


def _deg_call(dst32):
    @functools.partial(
        pl.kernel,
        out_type=jax.ShapeDtypeStruct((32, N), _f32),
        mesh=_vsmesh(),
        scratch_types=[
            pltpu.VMEM((EDGES_PER_W32,), jnp.int32),
            pltpu.VMEM((10016,), _f32),
        ],
        compiler_params=_sc_compiler_params(layout_passes=False),
    )
    def deg_kernel(dst_hbm, out_hbm, dstv, histv):
        c = lax.axis_index("c")
        s = lax.axis_index("s")
        w = c * NS + s
        pltpu.sync_copy(dst_hbm.at[w], dstv)
        zf = jnp.zeros((16,), _f32)
        onef = jnp.ones((16,), _f32)

        @pl.loop(0, 10016 // 16)
        def _(i):
            histv[pl.ds(i * 16, 16)] = zf

        @pl.loop(0, EDGES_PER_W32 // 16)
        def _(i):
            idx = dstv[pl.ds(i * 16, 16)]
            plsc.addupdate_scatter(histv, [idx], onef)

        pltpu.sync_copy(histv.at[pl.ds(0, N)], out_hbm.at[w])

    return deg_kernel(dst32)


def _agg_call(hs2, src_t, dst_t, f_half):
    @functools.partial(
        pl.kernel,
        out_type=jax.ShapeDtypeStruct((2, N, f_half), _f32),
        mesh=_vsmesh(),
        scratch_types=[
            pltpu.VMEM((NCHUNK, CHUNK), jnp.int32),
            pltpu.VMEM((NCHUNK, CHUNK), jnp.int32),
            pltpu.VMEM((CHUNK, f_half), _f32),
            pltpu.VMEM((CHUNK, f_half), _f32),
            pltpu.VMEM((CHUNK, f_half), _f32),
            pltpu.VMEM_SHARED((ACC_ROWS, f_half), _f32),
            pltpu.SemaphoreType.DMA,
            pltpu.SemaphoreType.DMA,
            pltpu.SemaphoreType.DMA,
            pltpu.SemaphoreType.DMA,
            pltpu.SemaphoreType.DMA,
            pltpu.SemaphoreType.DMA,
        ],
        compiler_params=_sc_compiler_params(),
    )
    def agg_kernel(hs_hbm, src_hbm, dst_hbm, out_hbm,
                   srcv, dstv, r0, r1, r2, acc,
                   g0, g1, g2, s0, s1, s2):
        c = lax.axis_index("c")
        s = lax.axis_index("s")
        hs = hs_hbm.at[c]
        out = out_hbm.at[c]
        pltpu.sync_copy(src_hbm.at[s], srcv)
        pltpu.sync_copy(dst_hbm.at[s], dstv)

        zf = jnp.zeros((16,), _f32)

        @pl.loop(0, CHUNK)
        def _(r):
            @pl.loop(0, f_half // 16)
            def _(q):
                r0[r, pl.ds(q * 16, 16)] = zf

        base = s * (ACC_ROWS // NS)

        @pl.loop(0, 7)
        def _(k):
            pltpu.sync_copy(r0, acc.at[pl.ds(base + k * CHUNK, CHUNK)])

        pltpu.sync_copy(r0.at[pl.ds(0, 66)],
                        acc.at[pl.ds(base + 7 * CHUNK, 66)])

        plsc.subcore_barrier()

        bufs = ((r0, g0, s0), (r1, g1, s1), (r2, g2, s2))
        for k, (r, g, _s) in enumerate(bufs):
            pltpu.async_copy(hs.at[srcv.at[k]], r, g)

        @pl.loop(0, NCHUNK // 3)
        def _(i):
            j0 = 3 * i
            for k, (r, g, ss) in enumerate(bufs):
                j = j0 + k
                pltpu.make_async_copy(hs.at[srcv.at[j]], r, g).wait()
                pltpu.async_copy(r, acc.at[dstv.at[j]], ss, add=True)
            for k, (r, g, ss) in enumerate(bufs):
                j = j0 + k
                pltpu.make_async_copy(r, acc.at[dstv.at[j]], ss).wait()

                @pl.when(j + 3 < NCHUNK)
                def _():
                    pltpu.async_copy(hs.at[srcv.at[j + 3]], r, g)

        plsc.subcore_barrier()
        pltpu.sync_copy(acc.at[pl.ds(s * ROWS_PER_TILE, ROWS_PER_TILE)],
                        out.at[pl.ds(s * ROWS_PER_TILE, ROWS_PER_TILE)])

    return agg_kernel(hs2, src_t, dst_t)


_DOT = functools.partial(
    lax.dot_general,
    precision=lax.Precision.HIGHEST,
    preferred_element_type=_f32,
)


def _mm_body(x_ref, w_ref, o_ref):
    o_ref[...] = _DOT(x_ref[...], w_ref[...], (((1,), (0,)), ((), ())))


def _mm_call(x, w):
    m, k = x.shape
    n = w.shape[1]
    return pl.pallas_call(
        _mm_body,
        grid=(m // BLK,),
        in_specs=[pl.BlockSpec((BLK, k), lambda i: (i, 0)),
                  pl.BlockSpec((k, n), lambda i: (0, 0))],
        out_specs=pl.BlockSpec((BLK, n), lambda i: (i, 0)),
        out_shape=jax.ShapeDtypeStruct((m, n), _f32),
    )(x, w)


def _dinv_body(p_ref, dv_ref):
    ones = jnp.ones((32, 1), _f32)
    deg = _DOT(p_ref[...], ones, (((0,), (0,)), ((), ()))) + 1.0
    dv_ref[...] = lax.rsqrt(deg)


def _dinv_call(partials):
    return pl.pallas_call(
        _dinv_body,
        in_specs=[pl.BlockSpec((32, N), lambda: (0, 0))],
        out_specs=pl.BlockSpec((N, 1), lambda: (0, 0)),
        out_shape=jax.ShapeDtypeStruct((N, 1), _f32),
    )(partials)


def _scale1_body(dv_ref, h_ref, o_ref):
    hs = h_ref[...] * dv_ref[...]
    o_ref[0] = hs[:, :HID // 2]
    o_ref[1] = hs[:, HID // 2:]


def _scale1_call(dinv, h1):
    return pl.pallas_call(
        _scale1_body,
        grid=(N // BLK,),
        in_specs=[pl.BlockSpec((BLK, 1), lambda i: (i, 0)),
                  pl.BlockSpec((BLK, HID), lambda i: (i, 0))],
        out_specs=pl.BlockSpec((2, BLK, HID // 2), lambda i: (0, i, 0)),
        out_shape=jax.ShapeDtypeStruct((2, N, HID // 2), _f32),
    )(dinv, h1)


def _layer_body(lo_ref, hi_ref, h1_ref, dv_ref, b1_ref, w2_ref,
                h2_ref, o2_ref):
    s1 = jnp.concatenate([lo_ref[0], hi_ref[0]], axis=1)
    dinv = dv_ref[...]
    out1 = dinv * s1 + (dinv * dinv) * h1_ref[...] + b1_ref[...]
    h = jnp.maximum(out1, 0.0)
    h2 = _DOT(h, w2_ref[...], (((1,), (0,)), ((), ())))
    h2_ref[...] = h2
    hs2 = dinv * h2
    o2_ref[0] = hs2[:, :F_OUT // 2]
    o2_ref[1] = hs2[:, F_OUT // 2:]


def _layer_call(s1, h1, dinv, b1, w2):
    return pl.pallas_call(
        _layer_body,
        grid=(N // BLK,),
        in_specs=[pl.BlockSpec((1, BLK, HID // 2), lambda i: (0, i, 0)),
                  pl.BlockSpec((1, BLK, HID // 2), lambda i: (1, i, 0)),
                  pl.BlockSpec((BLK, HID), lambda i: (i, 0)),
                  pl.BlockSpec((BLK, 1), lambda i: (i, 0)),
                  pl.BlockSpec((1, HID), lambda i: (0, 0)),
                  pl.BlockSpec((HID, F_OUT), lambda i: (0, 0))],
        out_specs=[pl.BlockSpec((BLK, F_OUT), lambda i: (i, 0)),
                   pl.BlockSpec((2, BLK, F_OUT // 2), lambda i: (0, i, 0))],
        out_shape=[jax.ShapeDtypeStruct((N, F_OUT), _f32),
                   jax.ShapeDtypeStruct((2, N, F_OUT // 2), _f32)],
    )(s1, s1, h1, dinv, b1, w2)


def _final_body(lo_ref, hi_ref, h2_ref, dv_ref, b2_ref, o_ref):
    s2 = jnp.concatenate([lo_ref[0], hi_ref[0]], axis=1)
    dinv = dv_ref[...]
    o_ref[...] = dinv * s2 + (dinv * dinv) * h2_ref[...] + b2_ref[...]


def _final_call(s2, h2, dinv, b2):
    return pl.pallas_call(
        _final_body,
        grid=(N // BLK,),
        in_specs=[pl.BlockSpec((1, BLK, F_OUT // 2), lambda i: (0, i, 0)),
                  pl.BlockSpec((1, BLK, F_OUT // 2), lambda i: (1, i, 0)),
                  pl.BlockSpec((BLK, F_OUT), lambda i: (i, 0)),
                  pl.BlockSpec((BLK, 1), lambda i: (i, 0)),
                  pl.BlockSpec((1, F_OUT), lambda i: (0, 0))],
        out_specs=pl.BlockSpec((BLK, F_OUT), lambda i: (i, 0)),
        out_shape=jax.ShapeDtypeStruct((N, F_OUT), _f32),
    )(s2, s2, h2, dinv, b2)


def kernel(x, edge_index, W1, b1, W2, b2):
    src = edge_index[0]
    dst = edge_index[1]
    pad = E_PAD - E
    src_p = jnp.concatenate([src, jnp.zeros((pad,), jnp.int32)])
    dst_p = jnp.concatenate([dst, jnp.full((pad,), DUMMY, jnp.int32)])
    src_t = src_p.reshape(NS, NCHUNK, CHUNK)
    dst_t = dst_p.reshape(NS, NCHUNK, CHUNK)
    dst32 = dst_p.reshape(32, EDGES_PER_W32)

    partials = _deg_call(dst32)
    h1 = _mm_call(x, W1)
    dinv = _dinv_call(partials)
    hs1 = _scale1_call(dinv, h1)
    s1 = _agg_call(hs1, src_t, dst_t, HID // 2)
    h2, hs2 = _layer_call(s1, h1, dinv, b1.reshape(1, HID), W2)
    s2 = _agg_call(hs2, src_t, dst_t, F_OUT // 2)
    return _final_call(s2, h2, dinv, b2.reshape(1, F_OUT))


# speedup vs baseline: 19.7309x; 1.4164x over previous
import dataclasses
import functools

import jax
import jax.numpy as jnp
from jax import lax
from jax.experimental import pallas as pl
from jax.experimental.pallas import tpu as pltpu
from jax.experimental.pallas import tpu_sc as plsc

N = 10000
E = 160000
F_IN = 256
HID = 256
F_OUT = 128

NS = 16
EDGES_PER_W32 = E // 32
ACC_ROWS = 10016
ROWS_PER_TILE = N // NS
BLK = 2000

_f32 = jnp.float32


def _vsmesh():
    return plsc.VectorSubcoreMesh(core_axis_name="c", subcore_axis_name="s")


def _sc_compiler_params(layout_passes=True):
    return pltpu.CompilerParams(
        use_tc_tiling_on_sc=False,
        needs_layout_passes=layout_passes,
        internal_scratch_in_bytes=0,
    )


def _deg_call(dst32):
    nfull = EDGES_PER_W32 // 16
    rem = EDGES_PER_W32 - nfull * 16

    @functools.partial(
        pl.kernel,
        out_type=jax.ShapeDtypeStruct((32, N), _f32),
        mesh=_vsmesh(),
        scratch_types=[
            pltpu.VMEM((EDGES_PER_W32 + 16,), jnp.int32),
            pltpu.VMEM((10016,), _f32),
        ],
        compiler_params=_sc_compiler_params(layout_passes=False),
    )
    def deg_kernel(dst_hbm, out_hbm, dstv, histv):
        c = lax.axis_index("c")
        s = lax.axis_index("s")
        w = c * NS + s
        dstv[pl.ds(EDGES_PER_W32 - rem, 16)] = jnp.zeros((16,), jnp.int32)
        pltpu.sync_copy(dst_hbm.at[w], dstv.at[pl.ds(0, EDGES_PER_W32)])
        zf = jnp.zeros((16,), _f32)
        onef = jnp.ones((16,), _f32)

        @pl.loop(0, 10016 // 16)
        def _(i):
            histv[pl.ds(i * 16, 16)] = zf

        @pl.loop(0, nfull)
        def _(i):
            idx = dstv[pl.ds(i * 16, 16)]
            plsc.addupdate_scatter(histv, [idx], onef)

        tail = dstv[pl.ds(nfull * 16, 16)]
        lane = lax.broadcasted_iota(jnp.int32, (16,), 0)
        plsc.addupdate_scatter(histv, [tail], onef, mask=lane < rem)

        pltpu.sync_copy(histv.at[pl.ds(0, N)], out_hbm.at[w])

    return deg_kernel(dst32)


def _agg_call(hs2, src_t, dst_t, feature_split):
    nt, nchunk, chunk = src_t.shape
    f = hs2.shape[-1]
    zslices = (ACC_ROWS // NS) // chunk
    zrem = (ACC_ROWS // NS) - zslices * chunk

    @functools.partial(
        pl.kernel,
        out_type=jax.ShapeDtypeStruct((2, N, f), _f32),
        mesh=_vsmesh(),
        scratch_types=[
            pltpu.VMEM((nchunk, chunk), jnp.int32),
            pltpu.VMEM((nchunk, chunk), jnp.int32),
            pltpu.VMEM((chunk, f), _f32),
            pltpu.VMEM((chunk, f), _f32),
            pltpu.VMEM((chunk, f), _f32),
            pltpu.VMEM_SHARED((ACC_ROWS, f), _f32),
            pltpu.SemaphoreType.DMA,
            pltpu.SemaphoreType.DMA,
            pltpu.SemaphoreType.DMA,
            pltpu.SemaphoreType.DMA,
            pltpu.SemaphoreType.DMA,
            pltpu.SemaphoreType.DMA,
        ],
        compiler_params=_sc_compiler_params(),
    )
    def agg_kernel(hs_hbm, src_hbm, dst_hbm, out_hbm,
                   srcv, dstv, r0, r1, r2, acc,
                   g0, g1, g2, s0, s1, s2):
        c = lax.axis_index("c")
        s = lax.axis_index("s")
        if feature_split:
            hs = hs_hbm.at[c]
            row = s
        else:
            hs = hs_hbm
            row = c * NS + s
        out = out_hbm.at[c]
        pltpu.sync_copy(src_hbm.at[row], srcv)
        pltpu.sync_copy(dst_hbm.at[row], dstv)

        zf = jnp.zeros((16,), _f32)

        @pl.loop(0, chunk)
        def _(r):
            @pl.loop(0, f // 16)
            def _(q):
                r0[r, pl.ds(q * 16, 16)] = zf

        base = s * (ACC_ROWS // NS)

        @pl.loop(0, zslices)
        def _(k):
            pltpu.sync_copy(r0, acc.at[pl.ds(base + k * chunk, chunk)])

        pltpu.sync_copy(r0.at[pl.ds(0, zrem)],
                        acc.at[pl.ds(base + zslices * chunk, zrem)])

        plsc.subcore_barrier()

        bufs = ((r0, g0, s0), (r1, g1, s1), (r2, g2, s2))
        for k, (r, g, _s) in enumerate(bufs):
            pltpu.async_copy(hs.at[srcv.at[k]], r, g)

        @pl.loop(0, nchunk // 3)
        def _(i):
            j0 = 3 * i
            for k, (r, g, ss) in enumerate(bufs):
                j = j0 + k
                pltpu.make_async_copy(hs.at[srcv.at[j]], r, g).wait()
                pltpu.async_copy(r, acc.at[dstv.at[j]], ss, add=True)
            for k, (r, g, ss) in enumerate(bufs):
                j = j0 + k
                pltpu.make_async_copy(r, acc.at[dstv.at[j]], ss).wait()

                @pl.when(j + 3 < nchunk)
                def _():
                    pltpu.async_copy(hs.at[srcv.at[j + 3]], r, g)

        for k in range(nchunk % 3):
            j = (nchunk // 3) * 3 + k
            r, g, ss = bufs[k]
            pltpu.make_async_copy(hs.at[srcv.at[j]], r, g).wait()
            pltpu.async_copy(r, acc.at[dstv.at[j]], ss, add=True)
        for k in range(nchunk % 3):
            j = (nchunk // 3) * 3 + k
            r, g, ss = bufs[k]
            pltpu.make_async_copy(r, acc.at[dstv.at[j]], ss).wait()

        plsc.subcore_barrier()
        pltpu.sync_copy(acc.at[pl.ds(s * ROWS_PER_TILE, ROWS_PER_TILE)],
                        out.at[pl.ds(s * ROWS_PER_TILE, ROWS_PER_TILE)])

    return agg_kernel(hs2, src_t, dst_t)


_DOT = functools.partial(
    lax.dot_general,
    precision=lax.Precision.HIGHEST,
    preferred_element_type=_f32,
)


def _mm_body(x_ref, w_ref, o_ref):
    o_ref[...] = _DOT(x_ref[...], w_ref[...], (((1,), (0,)), ((), ())))


def _mm_call(x, w):
    m, k = x.shape
    n = w.shape[1]
    return pl.pallas_call(
        _mm_body,
        grid=(m // BLK,),
        in_specs=[pl.BlockSpec((BLK, k), lambda i: (i, 0)),
                  pl.BlockSpec((k, n), lambda i: (0, 0))],
        out_specs=pl.BlockSpec((BLK, n), lambda i: (i, 0)),
        out_shape=jax.ShapeDtypeStruct((m, n), _f32),
    )(x, w)


def _dinv_body(p_ref, dv_ref):
    ones = jnp.ones((32, 1), _f32)
    deg = _DOT(p_ref[...], ones, (((0,), (0,)), ((), ()))) + 1.0
    dv_ref[...] = lax.rsqrt(deg)


def _dinv_call(partials):
    return pl.pallas_call(
        _dinv_body,
        in_specs=[pl.BlockSpec((32, N), lambda: (0, 0))],
        out_specs=pl.BlockSpec((N, 1), lambda: (0, 0)),
        out_shape=jax.ShapeDtypeStruct((N, 1), _f32),
    )(partials)


def _scale1_body(dv_ref, h_ref, o_ref):
    hs = h_ref[...] * dv_ref[...]
    o_ref[0] = hs[:, :HID // 2]
    o_ref[1] = hs[:, HID // 2:]


def _scale1_call(dinv, h1):
    return pl.pallas_call(
        _scale1_body,
        grid=(N // BLK,),
        in_specs=[pl.BlockSpec((BLK, 1), lambda i: (i, 0)),
                  pl.BlockSpec((BLK, HID), lambda i: (i, 0))],
        out_specs=pl.BlockSpec((2, BLK, HID // 2), lambda i: (0, i, 0)),
        out_shape=jax.ShapeDtypeStruct((2, N, HID // 2), _f32),
    )(dinv, h1)


def _layer_body(lo_ref, hi_ref, h1_ref, dv_ref, b1_ref, w2_ref,
                h2_ref, o2_ref):
    s1 = jnp.concatenate([lo_ref[0], hi_ref[0]], axis=1)
    dinv = dv_ref[...]
    out1 = dinv * s1 + (dinv * dinv) * h1_ref[...] + b1_ref[...]
    h = jnp.maximum(out1, 0.0)
    h2 = _DOT(h, w2_ref[...], (((1,), (0,)), ((), ())))
    h2_ref[...] = h2
    o2_ref[...] = dinv * h2


def _layer_call(s1, h1, dinv, b1, w2):
    return pl.pallas_call(
        _layer_body,
        grid=(N // BLK,),
        in_specs=[pl.BlockSpec((1, BLK, HID // 2), lambda i: (0, i, 0)),
                  pl.BlockSpec((1, BLK, HID // 2), lambda i: (1, i, 0)),
                  pl.BlockSpec((BLK, HID), lambda i: (i, 0)),
                  pl.BlockSpec((BLK, 1), lambda i: (i, 0)),
                  pl.BlockSpec((1, HID), lambda i: (0, 0)),
                  pl.BlockSpec((HID, F_OUT), lambda i: (0, 0))],
        out_specs=[pl.BlockSpec((BLK, F_OUT), lambda i: (i, 0)),
                   pl.BlockSpec((BLK, F_OUT), lambda i: (i, 0))],
        out_shape=[jax.ShapeDtypeStruct((N, F_OUT), _f32),
                   jax.ShapeDtypeStruct((N, F_OUT), _f32)],
    )(s1, s1, h1, dinv, b1, w2)


def _final_body(lo_ref, hi_ref, h2_ref, dv_ref, b2_ref, o_ref):
    s2 = lo_ref[0] + hi_ref[0]
    dinv = dv_ref[...]
    o_ref[...] = dinv * s2 + (dinv * dinv) * h2_ref[...] + b2_ref[...]


def _final_call(s2, h2, dinv, b2):
    return pl.pallas_call(
        _final_body,
        grid=(N // BLK,),
        in_specs=[pl.BlockSpec((1, BLK, F_OUT), lambda i: (0, i, 0)),
                  pl.BlockSpec((1, BLK, F_OUT), lambda i: (1, i, 0)),
                  pl.BlockSpec((BLK, F_OUT), lambda i: (i, 0)),
                  pl.BlockSpec((BLK, 1), lambda i: (i, 0)),
                  pl.BlockSpec((1, F_OUT), lambda i: (0, 0))],
        out_specs=pl.BlockSpec((BLK, F_OUT), lambda i: (i, 0)),
        out_shape=jax.ShapeDtypeStruct((N, F_OUT), _f32),
    )(s2, s2, h2, dinv, b2)


def kernel(x, edge_index, W1, b1, W2, b2):
    src = edge_index[0]
    dst = edge_index[1]
    src_t1 = src.reshape(NS, 125, 80)
    dst_t1 = dst.reshape(NS, 125, 80)
    src_t2 = src.reshape(2 * NS, 50, 100)
    dst_t2 = dst.reshape(2 * NS, 50, 100)
    dst32 = dst.reshape(32, EDGES_PER_W32)

    partials = _deg_call(dst32)
    h1 = _mm_call(x, W1)
    dinv = _dinv_call(partials)
    hs1 = _scale1_call(dinv, h1)
    s1 = _agg_call(hs1, src_t1, dst_t1, True)
    h2, hs2 = _layer_call(s1, h1, dinv, b1.reshape(1, HID), W2)
    s2 = _agg_call(hs2, src_t2, dst_t2, False)
    return _final_call(s2, h2, dinv, b2.reshape(1, F_OUT))


# speedup vs baseline: 20.4755x; 1.0377x over previous
import dataclasses
import functools

import jax
import jax.numpy as jnp
from jax import lax
from jax.experimental import pallas as pl
from jax.experimental.pallas import tpu as pltpu
from jax.experimental.pallas import tpu_sc as plsc

N = 10000
E = 160000
F_IN = 256
HID = 256
F_OUT = 128

NS = 16
EDGES_PER_W32 = E // 32
ACC_ROWS = 10016
ROWS_PER_TILE = N // NS
BLK = 2000

_f32 = jnp.float32


def _vsmesh():
    return plsc.VectorSubcoreMesh(core_axis_name="c", subcore_axis_name="s")


def _sc_compiler_params(layout_passes=True):
    return pltpu.CompilerParams(
        use_tc_tiling_on_sc=False,
        needs_layout_passes=layout_passes,
        internal_scratch_in_bytes=0,
    )


def _deg_call(dst32):
    nfull = EDGES_PER_W32 // 16
    rem = EDGES_PER_W32 - nfull * 16

    @functools.partial(
        pl.kernel,
        out_type=jax.ShapeDtypeStruct((32, N), _f32),
        mesh=_vsmesh(),
        scratch_types=[
            pltpu.VMEM((EDGES_PER_W32 + 16,), jnp.int32),
            pltpu.VMEM((10016,), _f32),
        ],
        compiler_params=_sc_compiler_params(layout_passes=False),
    )
    def deg_kernel(dst_hbm, out_hbm, dstv, histv):
        c = lax.axis_index("c")
        s = lax.axis_index("s")
        w = c * NS + s
        dstv[pl.ds(EDGES_PER_W32 - rem, 16)] = jnp.zeros((16,), jnp.int32)
        pltpu.sync_copy(dst_hbm.at[w], dstv.at[pl.ds(0, EDGES_PER_W32)])
        zf = jnp.zeros((16,), _f32)
        onef = jnp.ones((16,), _f32)

        @pl.loop(0, 10016 // 16)
        def _(i):
            histv[pl.ds(i * 16, 16)] = zf

        @pl.loop(0, nfull)
        def _(i):
            idx = dstv[pl.ds(i * 16, 16)]
            plsc.addupdate_scatter(histv, [idx], onef)

        tail = dstv[pl.ds(nfull * 16, 16)]
        lane = lax.broadcasted_iota(jnp.int32, (16,), 0)
        plsc.addupdate_scatter(histv, [tail], onef, mask=lane < rem)

        pltpu.sync_copy(histv.at[pl.ds(0, N)], out_hbm.at[w])

    return deg_kernel(dst32)


def _agg_call(hs2, src_t, dst_t, feature_split):
    nt, nchunk, chunk = src_t.shape
    f = hs2.shape[-1]
    zslices = (ACC_ROWS // NS) // chunk
    zrem = (ACC_ROWS // NS) - zslices * chunk

    @functools.partial(
        pl.kernel,
        out_type=jax.ShapeDtypeStruct((2, N, f), _f32),
        mesh=_vsmesh(),
        scratch_types=[
            pltpu.VMEM((nchunk, chunk), jnp.int32),
            pltpu.VMEM((nchunk, chunk), jnp.int32),
            pltpu.VMEM((chunk, f), _f32),
            pltpu.VMEM((chunk, f), _f32),
            pltpu.VMEM((chunk, f), _f32),
            pltpu.VMEM_SHARED((ACC_ROWS, f), _f32),
            pltpu.SemaphoreType.DMA,
            pltpu.SemaphoreType.DMA,
            pltpu.SemaphoreType.DMA,
            pltpu.SemaphoreType.DMA,
            pltpu.SemaphoreType.DMA,
            pltpu.SemaphoreType.DMA,
        ],
        compiler_params=_sc_compiler_params(),
    )
    def agg_kernel(hs_hbm, src_hbm, dst_hbm, out_hbm,
                   srcv, dstv, r0, r1, r2, acc,
                   g0, g1, g2, s0, s1, s2):
        c = lax.axis_index("c")
        s = lax.axis_index("s")
        if feature_split:
            hs = hs_hbm.at[c]
            row = s
        else:
            hs = hs_hbm
            row = c * NS + s
        out = out_hbm.at[c]
        pltpu.sync_copy(src_hbm.at[row], srcv)
        pltpu.sync_copy(dst_hbm.at[row], dstv)

        zf = jnp.zeros((16,), _f32)

        @pl.loop(0, chunk)
        def _(r):
            @pl.loop(0, f // 16)
            def _(q):
                r0[r, pl.ds(q * 16, 16)] = zf

        base = s * (ACC_ROWS // NS)

        @pl.loop(0, zslices)
        def _(k):
            pltpu.sync_copy(r0, acc.at[pl.ds(base + k * chunk, chunk)])

        pltpu.sync_copy(r0.at[pl.ds(0, zrem)],
                        acc.at[pl.ds(base + zslices * chunk, zrem)])

        plsc.subcore_barrier()

        bufs = ((r0, g0, s0), (r1, g1, s1), (r2, g2, s2))
        for k, (r, g, _s) in enumerate(bufs):
            pltpu.async_copy(hs.at[srcv.at[k]], r, g)

        @pl.loop(0, nchunk // 3)
        def _(i):
            j0 = 3 * i
            for k, (r, g, ss) in enumerate(bufs):
                j = j0 + k
                pltpu.make_async_copy(hs.at[srcv.at[j]], r, g).wait()
                pltpu.async_copy(r, acc.at[dstv.at[j]], ss, add=True)
            for k, (r, g, ss) in enumerate(bufs):
                j = j0 + k
                pltpu.make_async_copy(r, acc.at[dstv.at[j]], ss).wait()

                @pl.when(j + 3 < nchunk)
                def _():
                    pltpu.async_copy(hs.at[srcv.at[j + 3]], r, g)

        for k in range(nchunk % 3):
            j = (nchunk // 3) * 3 + k
            r, g, ss = bufs[k]
            pltpu.make_async_copy(hs.at[srcv.at[j]], r, g).wait()
            pltpu.async_copy(r, acc.at[dstv.at[j]], ss, add=True)
        for k in range(nchunk % 3):
            j = (nchunk // 3) * 3 + k
            r, g, ss = bufs[k]
            pltpu.make_async_copy(r, acc.at[dstv.at[j]], ss).wait()

        plsc.subcore_barrier()
        pltpu.sync_copy(acc.at[pl.ds(s * ROWS_PER_TILE, ROWS_PER_TILE)],
                        out.at[pl.ds(s * ROWS_PER_TILE, ROWS_PER_TILE)])

    return agg_kernel(hs2, src_t, dst_t)


_DOT = functools.partial(
    lax.dot_general,
    precision=lax.Precision.DEFAULT,
    preferred_element_type=_f32,
)


def _mm_body(x_ref, w_ref, o_ref):
    o_ref[...] = _DOT(x_ref[...], w_ref[...], (((1,), (0,)), ((), ())))


def _mm_call(x, w):
    m, k = x.shape
    n = w.shape[1]
    return pl.pallas_call(
        _mm_body,
        grid=(m // BLK,),
        in_specs=[pl.BlockSpec((BLK, k), lambda i: (i, 0)),
                  pl.BlockSpec((k, n), lambda i: (0, 0))],
        out_specs=pl.BlockSpec((BLK, n), lambda i: (i, 0)),
        out_shape=jax.ShapeDtypeStruct((m, n), _f32),
    )(x, w)


def _dinv_body(p_ref, dv_ref):
    ones = jnp.ones((32, 1), _f32)
    deg = _DOT(p_ref[...], ones, (((0,), (0,)), ((), ()))) + 1.0
    dv_ref[...] = lax.rsqrt(deg)


def _dinv_call(partials):
    return pl.pallas_call(
        _dinv_body,
        in_specs=[pl.BlockSpec((32, N), lambda: (0, 0))],
        out_specs=pl.BlockSpec((N, 1), lambda: (0, 0)),
        out_shape=jax.ShapeDtypeStruct((N, 1), _f32),
    )(partials)


def _scale1_body(dv_ref, h_ref, o_ref):
    hs = h_ref[...] * dv_ref[...]
    o_ref[0] = hs[:, :HID // 2]
    o_ref[1] = hs[:, HID // 2:]


def _scale1_call(dinv, h1):
    return pl.pallas_call(
        _scale1_body,
        grid=(N // BLK,),
        in_specs=[pl.BlockSpec((BLK, 1), lambda i: (i, 0)),
                  pl.BlockSpec((BLK, HID), lambda i: (i, 0))],
        out_specs=pl.BlockSpec((2, BLK, HID // 2), lambda i: (0, i, 0)),
        out_shape=jax.ShapeDtypeStruct((2, N, HID // 2), _f32),
    )(dinv, h1)


def _layer_body(lo_ref, hi_ref, h1_ref, dv_ref, b1_ref, w2_ref,
                h2_ref, o2_ref):
    s1 = jnp.concatenate([lo_ref[0], hi_ref[0]], axis=1)
    dinv = dv_ref[...]
    out1 = dinv * s1 + (dinv * dinv) * h1_ref[...] + b1_ref[...]
    h = jnp.maximum(out1, 0.0)
    h2 = _DOT(h, w2_ref[...], (((1,), (0,)), ((), ())))
    h2_ref[...] = h2
    o2_ref[...] = dinv * h2


def _layer_call(s1, h1, dinv, b1, w2):
    return pl.pallas_call(
        _layer_body,
        grid=(N // BLK,),
        in_specs=[pl.BlockSpec((1, BLK, HID // 2), lambda i: (0, i, 0)),
                  pl.BlockSpec((1, BLK, HID // 2), lambda i: (1, i, 0)),
                  pl.BlockSpec((BLK, HID), lambda i: (i, 0)),
                  pl.BlockSpec((BLK, 1), lambda i: (i, 0)),
                  pl.BlockSpec((1, HID), lambda i: (0, 0)),
                  pl.BlockSpec((HID, F_OUT), lambda i: (0, 0))],
        out_specs=[pl.BlockSpec((BLK, F_OUT), lambda i: (i, 0)),
                   pl.BlockSpec((BLK, F_OUT), lambda i: (i, 0))],
        out_shape=[jax.ShapeDtypeStruct((N, F_OUT), _f32),
                   jax.ShapeDtypeStruct((N, F_OUT), _f32)],
    )(s1, s1, h1, dinv, b1, w2)


def _final_body(lo_ref, hi_ref, h2_ref, dv_ref, b2_ref, o_ref):
    s2 = lo_ref[0] + hi_ref[0]
    dinv = dv_ref[...]
    o_ref[...] = dinv * s2 + (dinv * dinv) * h2_ref[...] + b2_ref[...]


def _final_call(s2, h2, dinv, b2):
    return pl.pallas_call(
        _final_body,
        grid=(N // BLK,),
        in_specs=[pl.BlockSpec((1, BLK, F_OUT), lambda i: (0, i, 0)),
                  pl.BlockSpec((1, BLK, F_OUT), lambda i: (1, i, 0)),
                  pl.BlockSpec((BLK, F_OUT), lambda i: (i, 0)),
                  pl.BlockSpec((BLK, 1), lambda i: (i, 0)),
                  pl.BlockSpec((1, F_OUT), lambda i: (0, 0))],
        out_specs=pl.BlockSpec((BLK, F_OUT), lambda i: (i, 0)),
        out_shape=jax.ShapeDtypeStruct((N, F_OUT), _f32),
    )(s2, s2, h2, dinv, b2)


def kernel(x, edge_index, W1, b1, W2, b2):
    src = edge_index[0]
    dst = edge_index[1]
    src_t1 = src.reshape(NS, 125, 80)
    dst_t1 = dst.reshape(NS, 125, 80)
    src_t2 = src.reshape(2 * NS, 50, 100)
    dst_t2 = dst.reshape(2 * NS, 50, 100)
    dst32 = dst.reshape(32, EDGES_PER_W32)

    partials = _deg_call(dst32)
    h1 = _mm_call(x, W1)
    dinv = _dinv_call(partials)
    hs1 = _scale1_call(dinv, h1)
    s1 = _agg_call(hs1, src_t1, dst_t1, True)
    h2, hs2 = _layer_call(s1, h1, dinv, b1.reshape(1, HID), W2)
    s2 = _agg_call(hs2, src_t2, dst_t2, False)
    return _final_call(s2, h2, dinv, b2.reshape(1, F_OUT))


# speedup vs baseline: 20.9511x; 1.0232x over previous
import dataclasses
import functools

import jax
import jax.numpy as jnp
from jax import lax
from jax.experimental import pallas as pl
from jax.experimental.pallas import tpu as pltpu
from jax.experimental.pallas import tpu_sc as plsc

N = 10000
E = 160000
F_IN = 256
HID = 256
F_OUT = 128

NS = 16
EDGES_PER_W32 = E // 32
ACC_ROWS = 10016
ROWS_PER_TILE = N // NS
BLK = 2000

_f32 = jnp.float32


def _vsmesh():
    return plsc.VectorSubcoreMesh(core_axis_name="c", subcore_axis_name="s")


def _sc_compiler_params(layout_passes=True):
    return pltpu.CompilerParams(
        use_tc_tiling_on_sc=False,
        needs_layout_passes=layout_passes,
        internal_scratch_in_bytes=0,
    )


def _deg_call(dst32):
    nfull = EDGES_PER_W32 // 16
    rem = EDGES_PER_W32 - nfull * 16

    @functools.partial(
        pl.kernel,
        out_type=jax.ShapeDtypeStruct((32, N), _f32),
        mesh=_vsmesh(),
        scratch_types=[
            pltpu.VMEM((EDGES_PER_W32 + 16,), jnp.int32),
            pltpu.VMEM((10016,), _f32),
        ],
        compiler_params=_sc_compiler_params(layout_passes=False),
    )
    def deg_kernel(dst_hbm, out_hbm, dstv, histv):
        c = lax.axis_index("c")
        s = lax.axis_index("s")
        w = c * NS + s
        dstv[pl.ds(EDGES_PER_W32 - rem, 16)] = jnp.zeros((16,), jnp.int32)
        pltpu.sync_copy(dst_hbm.at[w], dstv.at[pl.ds(0, EDGES_PER_W32)])
        zf = jnp.zeros((16,), _f32)
        onef = jnp.ones((16,), _f32)

        @pl.loop(0, 10016 // 16)
        def _(i):
            histv[pl.ds(i * 16, 16)] = zf

        @pl.loop(0, nfull)
        def _(i):
            idx = dstv[pl.ds(i * 16, 16)]
            plsc.addupdate_scatter(histv, [idx], onef)

        tail = dstv[pl.ds(nfull * 16, 16)]
        lane = lax.broadcasted_iota(jnp.int32, (16,), 0)
        plsc.addupdate_scatter(histv, [tail], onef, mask=lane < rem)

        pltpu.sync_copy(histv.at[pl.ds(0, N)], out_hbm.at[w])

    return deg_kernel(dst32)


def _agg_call(hs2, src_t, dst_t, feature_split):
    nt, nchunk, chunk = src_t.shape
    f = hs2.shape[-1]
    zslices = (ACC_ROWS // NS) // chunk
    zrem = (ACC_ROWS // NS) - zslices * chunk

    @functools.partial(
        pl.kernel,
        out_type=jax.ShapeDtypeStruct((2, N, f), _f32),
        mesh=_vsmesh(),
        scratch_types=[
            pltpu.VMEM((nchunk, chunk), jnp.int32),
            pltpu.VMEM((nchunk, chunk), jnp.int32),
            pltpu.VMEM((chunk, f), _f32),
            pltpu.VMEM((chunk, f), _f32),
            pltpu.VMEM((chunk, f), _f32),
            pltpu.VMEM_SHARED((ACC_ROWS, f), _f32),
            pltpu.SemaphoreType.DMA,
            pltpu.SemaphoreType.DMA,
            pltpu.SemaphoreType.DMA,
            pltpu.SemaphoreType.DMA,
            pltpu.SemaphoreType.DMA,
            pltpu.SemaphoreType.DMA,
        ],
        compiler_params=_sc_compiler_params(),
    )
    def agg_kernel(hs_hbm, src_hbm, dst_hbm, out_hbm,
                   srcv, dstv, r0, r1, r2, acc,
                   g0, g1, g2, s0, s1, s2):
        c = lax.axis_index("c")
        s = lax.axis_index("s")
        if feature_split:
            hs = hs_hbm.at[c]
            row = s
        else:
            hs = hs_hbm
            row = c * NS + s
        out = out_hbm.at[c]
        pltpu.sync_copy(src_hbm.at[row], srcv)
        pltpu.sync_copy(dst_hbm.at[row], dstv)

        zf = jnp.zeros((16,), _f32)

        @pl.loop(0, chunk)
        def _(r):
            @pl.loop(0, f // 16)
            def _(q):
                r0[r, pl.ds(q * 16, 16)] = zf

        base = s * (ACC_ROWS // NS)

        @pl.loop(0, zslices)
        def _(k):
            pltpu.sync_copy(r0, acc.at[pl.ds(base + k * chunk, chunk)])

        pltpu.sync_copy(r0.at[pl.ds(0, zrem)],
                        acc.at[pl.ds(base + zslices * chunk, zrem)])

        plsc.subcore_barrier()

        bufs = ((r0, g0, s0), (r1, g1, s1), (r2, g2, s2))
        for k, (r, g, _s) in enumerate(bufs):
            pltpu.async_copy(hs.at[srcv.at[k]], r, g)

        @pl.loop(0, nchunk // 3)
        def _(i):
            j0 = 3 * i
            for k, (r, g, ss) in enumerate(bufs):
                j = j0 + k
                pltpu.make_async_copy(hs.at[srcv.at[j]], r, g).wait()
                pltpu.async_copy(r, acc.at[dstv.at[j]], ss, add=True)
            for k, (r, g, ss) in enumerate(bufs):
                j = j0 + k
                pltpu.make_async_copy(r, acc.at[dstv.at[j]], ss).wait()

                @pl.when(j + 3 < nchunk)
                def _():
                    pltpu.async_copy(hs.at[srcv.at[j + 3]], r, g)

        for k in range(nchunk % 3):
            j = (nchunk // 3) * 3 + k
            r, g, ss = bufs[k]
            pltpu.make_async_copy(hs.at[srcv.at[j]], r, g).wait()
            pltpu.async_copy(r, acc.at[dstv.at[j]], ss, add=True)
        for k in range(nchunk % 3):
            j = (nchunk // 3) * 3 + k
            r, g, ss = bufs[k]
            pltpu.make_async_copy(r, acc.at[dstv.at[j]], ss).wait()

        plsc.subcore_barrier()
        pltpu.sync_copy(acc.at[pl.ds(s * ROWS_PER_TILE, ROWS_PER_TILE)],
                        out.at[pl.ds(s * ROWS_PER_TILE, ROWS_PER_TILE)])

    return agg_kernel(hs2, src_t, dst_t)


_DOT = functools.partial(
    lax.dot_general,
    precision=lax.Precision.DEFAULT,
    preferred_element_type=_f32,
)


def _mmscale_body(dv_ref, x_ref, w_ref, o_ref):
    hs = _DOT(x_ref[...], w_ref[...], (((1,), (0,)), ((), ()))) * dv_ref[...]
    o_ref[0] = hs[:, :HID // 2]
    o_ref[1] = hs[:, HID // 2:]


def _mmscale_call(dinv, x, w):
    return pl.pallas_call(
        _mmscale_body,
        grid=(N // BLK,),
        in_specs=[pl.BlockSpec((BLK, 1), lambda i: (i, 0)),
                  pl.BlockSpec((BLK, F_IN), lambda i: (i, 0)),
                  pl.BlockSpec((F_IN, HID), lambda i: (0, 0))],
        out_specs=pl.BlockSpec((2, BLK, HID // 2), lambda i: (0, i, 0)),
        out_shape=jax.ShapeDtypeStruct((2, N, HID // 2), _f32),
    )(dinv, x, w)


def _dinv_body(p_ref, dv_ref):
    ones = jnp.ones((32, 1), _f32)
    deg = _DOT(p_ref[...], ones, (((0,), (0,)), ((), ()))) + 1.0
    dv_ref[...] = lax.rsqrt(deg)


def _dinv_call(partials):
    return pl.pallas_call(
        _dinv_body,
        in_specs=[pl.BlockSpec((32, N), lambda: (0, 0))],
        out_specs=pl.BlockSpec((N, 1), lambda: (0, 0)),
        out_shape=jax.ShapeDtypeStruct((N, 1), _f32),
    )(partials)


def _layer_body(lo_ref, hi_ref, hs1lo_ref, hs1hi_ref, dv_ref, b1_ref,
                w2_ref, o2_ref):
    s1 = jnp.concatenate([lo_ref[0] + hs1lo_ref[0],
                          hi_ref[0] + hs1hi_ref[0]], axis=1)
    dinv = dv_ref[...]
    out1 = dinv * s1 + b1_ref[...]
    h = jnp.maximum(out1, 0.0)
    h2 = _DOT(h, w2_ref[...], (((1,), (0,)), ((), ())))
    o2_ref[...] = dinv * h2


def _layer_call(s1, hs1, dinv, b1, w2):
    return pl.pallas_call(
        _layer_body,
        grid=(N // BLK,),
        in_specs=[pl.BlockSpec((1, BLK, HID // 2), lambda i: (0, i, 0)),
                  pl.BlockSpec((1, BLK, HID // 2), lambda i: (1, i, 0)),
                  pl.BlockSpec((1, BLK, HID // 2), lambda i: (0, i, 0)),
                  pl.BlockSpec((1, BLK, HID // 2), lambda i: (1, i, 0)),
                  pl.BlockSpec((BLK, 1), lambda i: (i, 0)),
                  pl.BlockSpec((1, HID), lambda i: (0, 0)),
                  pl.BlockSpec((HID, F_OUT), lambda i: (0, 0))],
        out_specs=pl.BlockSpec((BLK, F_OUT), lambda i: (i, 0)),
        out_shape=jax.ShapeDtypeStruct((N, F_OUT), _f32),
    )(s1, s1, hs1, hs1, dinv, b1, w2)


def _final_body(lo_ref, hi_ref, hs2_ref, dv_ref, b2_ref, o_ref):
    s2 = lo_ref[0] + hi_ref[0] + hs2_ref[...]
    o_ref[...] = dv_ref[...] * s2 + b2_ref[...]


def _final_call(s2, hs2, dinv, b2):
    return pl.pallas_call(
        _final_body,
        grid=(N // BLK,),
        in_specs=[pl.BlockSpec((1, BLK, F_OUT), lambda i: (0, i, 0)),
                  pl.BlockSpec((1, BLK, F_OUT), lambda i: (1, i, 0)),
                  pl.BlockSpec((BLK, F_OUT), lambda i: (i, 0)),
                  pl.BlockSpec((BLK, 1), lambda i: (i, 0)),
                  pl.BlockSpec((1, F_OUT), lambda i: (0, 0))],
        out_specs=pl.BlockSpec((BLK, F_OUT), lambda i: (i, 0)),
        out_shape=jax.ShapeDtypeStruct((N, F_OUT), _f32),
    )(s2, s2, hs2, dinv, b2)


def kernel(x, edge_index, W1, b1, W2, b2):
    src = edge_index[0]
    dst = edge_index[1]
    src_t1 = src.reshape(NS, 125, 80)
    dst_t1 = dst.reshape(NS, 125, 80)
    src_t2 = src.reshape(2 * NS, 50, 100)
    dst_t2 = dst.reshape(2 * NS, 50, 100)
    dst32 = dst.reshape(32, EDGES_PER_W32)

    partials = _deg_call(dst32)
    dinv = _dinv_call(partials)
    hs1 = _mmscale_call(dinv, x, W1)
    s1 = _agg_call(hs1, src_t1, dst_t1, True)
    hs2 = _layer_call(s1, hs1, dinv, b1.reshape(1, HID), W2)
    s2 = _agg_call(hs2, src_t2, dst_t2, False)
    return _final_call(s2, hs2, dinv, b2.reshape(1, F_OUT))


# speedup vs baseline: 21.2421x; 1.0139x over previous
import dataclasses
import functools

import jax
import jax.numpy as jnp
from jax import lax
from jax.experimental import pallas as pl
from jax.experimental.pallas import tpu as pltpu
from jax.experimental.pallas import tpu_sc as plsc

N = 10000
E = 160000
F_IN = 256
HID = 256
F_OUT = 128

NS = 16
EDGES_PER_W32 = E // 32
ACC_ROWS = 10016
ROWS_PER_TILE = N // NS
BLK = 2000

_f32 = jnp.float32


def _vsmesh():
    return plsc.VectorSubcoreMesh(core_axis_name="c", subcore_axis_name="s")


def _sc_compiler_params(layout_passes=True):
    return pltpu.CompilerParams(
        use_tc_tiling_on_sc=False,
        needs_layout_passes=layout_passes,
        internal_scratch_in_bytes=0,
    )


def _deg_call(edge_index):
    nfull = EDGES_PER_W32 // 16
    rem = EDGES_PER_W32 - nfull * 16

    @functools.partial(
        pl.kernel,
        out_type=jax.ShapeDtypeStruct((32, N), _f32),
        mesh=_vsmesh(),
        scratch_types=[
            pltpu.VMEM((EDGES_PER_W32 + 16,), jnp.int32),
            pltpu.VMEM((10016,), _f32),
        ],
        compiler_params=_sc_compiler_params(layout_passes=False),
    )
    def deg_kernel(edges_hbm, out_hbm, dstv, histv):
        c = lax.axis_index("c")
        s = lax.axis_index("s")
        w = c * NS + s
        dstv[pl.ds(EDGES_PER_W32 - rem, 16)] = jnp.zeros((16,), jnp.int32)
        pltpu.sync_copy(edges_hbm.at[1].at[pl.ds(w * EDGES_PER_W32,
                                                 EDGES_PER_W32)],
                        dstv.at[pl.ds(0, EDGES_PER_W32)])
        zf = jnp.zeros((16,), _f32)
        onef = jnp.ones((16,), _f32)

        @pl.loop(0, 10016 // 16)
        def _(i):
            histv[pl.ds(i * 16, 16)] = zf

        @pl.loop(0, nfull)
        def _(i):
            idx = dstv[pl.ds(i * 16, 16)]
            plsc.addupdate_scatter(histv, [idx], onef)

        tail = dstv[pl.ds(nfull * 16, 16)]
        lane = lax.broadcasted_iota(jnp.int32, (16,), 0)
        plsc.addupdate_scatter(histv, [tail], onef, mask=lane < rem)

        pltpu.sync_copy(histv.at[pl.ds(0, N)], out_hbm.at[w])

    return deg_kernel(edge_index)


def _agg_call(hs2, src_t, dst_t, feature_split):
    nt, nchunk, chunk = src_t.shape
    f = hs2.shape[-1]
    zslices = (ACC_ROWS // NS) // chunk
    zrem = (ACC_ROWS // NS) - zslices * chunk

    @functools.partial(
        pl.kernel,
        out_type=jax.ShapeDtypeStruct((2, N, f), _f32),
        mesh=_vsmesh(),
        scratch_types=[
            pltpu.VMEM((nchunk, chunk), jnp.int32),
            pltpu.VMEM((nchunk, chunk), jnp.int32),
            pltpu.VMEM((chunk, f), _f32),
            pltpu.VMEM((chunk, f), _f32),
            pltpu.VMEM((chunk, f), _f32),
            pltpu.VMEM_SHARED((ACC_ROWS, f), _f32),
            pltpu.SemaphoreType.DMA,
            pltpu.SemaphoreType.DMA,
            pltpu.SemaphoreType.DMA,
            pltpu.SemaphoreType.DMA,
            pltpu.SemaphoreType.DMA,
            pltpu.SemaphoreType.DMA,
        ],
        compiler_params=_sc_compiler_params(),
    )
    def agg_kernel(hs_hbm, src_hbm, dst_hbm, out_hbm,
                   srcv, dstv, r0, r1, r2, acc,
                   g0, g1, g2, s0, s1, s2):
        c = lax.axis_index("c")
        s = lax.axis_index("s")
        if feature_split:
            hs = hs_hbm.at[c]
            row = s
        else:
            hs = hs_hbm
            row = c * NS + s
        out = out_hbm.at[c]
        pltpu.sync_copy(src_hbm.at[row], srcv)
        pltpu.sync_copy(dst_hbm.at[row], dstv)

        zf = jnp.zeros((16,), _f32)

        @pl.loop(0, chunk)
        def _(r):
            @pl.loop(0, f // 16)
            def _(q):
                r0[r, pl.ds(q * 16, 16)] = zf

        base = s * (ACC_ROWS // NS)

        @pl.loop(0, zslices)
        def _(k):
            pltpu.sync_copy(r0, acc.at[pl.ds(base + k * chunk, chunk)])

        pltpu.sync_copy(r0.at[pl.ds(0, zrem)],
                        acc.at[pl.ds(base + zslices * chunk, zrem)])

        plsc.subcore_barrier()

        bufs = ((r0, g0, s0), (r1, g1, s1), (r2, g2, s2))
        for k, (r, g, _s) in enumerate(bufs):
            pltpu.async_copy(hs.at[srcv.at[k]], r, g)

        @pl.loop(0, nchunk // 3)
        def _(i):
            j0 = 3 * i
            for k, (r, g, ss) in enumerate(bufs):
                j = j0 + k
                pltpu.make_async_copy(hs.at[srcv.at[j]], r, g).wait()
                pltpu.async_copy(r, acc.at[dstv.at[j]], ss, add=True)
            for k, (r, g, ss) in enumerate(bufs):
                j = j0 + k
                pltpu.make_async_copy(r, acc.at[dstv.at[j]], ss).wait()

                @pl.when(j + 3 < nchunk)
                def _():
                    pltpu.async_copy(hs.at[srcv.at[j + 3]], r, g)

        for k in range(nchunk % 3):
            j = (nchunk // 3) * 3 + k
            r, g, ss = bufs[k]
            pltpu.make_async_copy(hs.at[srcv.at[j]], r, g).wait()
            pltpu.async_copy(r, acc.at[dstv.at[j]], ss, add=True)
        for k in range(nchunk % 3):
            j = (nchunk // 3) * 3 + k
            r, g, ss = bufs[k]
            pltpu.make_async_copy(r, acc.at[dstv.at[j]], ss).wait()

        plsc.subcore_barrier()
        pltpu.sync_copy(acc.at[pl.ds(s * ROWS_PER_TILE, ROWS_PER_TILE)],
                        out.at[pl.ds(s * ROWS_PER_TILE, ROWS_PER_TILE)])

    return agg_kernel(hs2, src_t, dst_t)


_DOT = functools.partial(
    lax.dot_general,
    precision=lax.Precision.DEFAULT,
    preferred_element_type=_f32,
)


def _mmscale_body(dv_ref, x_ref, w_ref, o_ref):
    hs = _DOT(x_ref[...], w_ref[...], (((1,), (0,)), ((), ()))) * dv_ref[...]
    o_ref[0] = hs[:, :HID // 2]
    o_ref[1] = hs[:, HID // 2:]


def _mmscale_call(dinv, x, w):
    return pl.pallas_call(
        _mmscale_body,
        grid=(N // BLK,),
        in_specs=[pl.BlockSpec((BLK, 1), lambda i: (i, 0)),
                  pl.BlockSpec((BLK, F_IN), lambda i: (i, 0)),
                  pl.BlockSpec((F_IN, HID), lambda i: (0, 0))],
        out_specs=pl.BlockSpec((2, BLK, HID // 2), lambda i: (0, i, 0)),
        out_shape=jax.ShapeDtypeStruct((2, N, HID // 2), _f32),
    )(dinv, x, w)


def _dinv_body(p_ref, dv_ref):
    ones = jnp.ones((32, 1), _f32)
    deg = _DOT(p_ref[...], ones, (((0,), (0,)), ((), ()))) + 1.0
    dv_ref[...] = lax.rsqrt(deg)


def _dinv_call(partials):
    return pl.pallas_call(
        _dinv_body,
        in_specs=[pl.BlockSpec((32, N), lambda: (0, 0))],
        out_specs=pl.BlockSpec((N, 1), lambda: (0, 0)),
        out_shape=jax.ShapeDtypeStruct((N, 1), _f32),
    )(partials)


def _layer_body(lo_ref, hi_ref, hs1lo_ref, hs1hi_ref, dv_ref, b1_ref,
                w2_ref, o2_ref):
    s1 = jnp.concatenate([lo_ref[0] + hs1lo_ref[0],
                          hi_ref[0] + hs1hi_ref[0]], axis=1)
    dinv = dv_ref[...]
    out1 = dinv * s1 + b1_ref[...]
    h = jnp.maximum(out1, 0.0)
    h2 = _DOT(h, w2_ref[...], (((1,), (0,)), ((), ())))
    o2_ref[...] = dinv * h2


def _layer_call(s1, hs1, dinv, b1, w2):
    return pl.pallas_call(
        _layer_body,
        grid=(N // BLK,),
        in_specs=[pl.BlockSpec((1, BLK, HID // 2), lambda i: (0, i, 0)),
                  pl.BlockSpec((1, BLK, HID // 2), lambda i: (1, i, 0)),
                  pl.BlockSpec((1, BLK, HID // 2), lambda i: (0, i, 0)),
                  pl.BlockSpec((1, BLK, HID // 2), lambda i: (1, i, 0)),
                  pl.BlockSpec((BLK, 1), lambda i: (i, 0)),
                  pl.BlockSpec((1, HID), lambda i: (0, 0)),
                  pl.BlockSpec((HID, F_OUT), lambda i: (0, 0))],
        out_specs=pl.BlockSpec((BLK, F_OUT), lambda i: (i, 0)),
        out_shape=jax.ShapeDtypeStruct((N, F_OUT), _f32),
    )(s1, s1, hs1, hs1, dinv, b1, w2)


def _final_body(lo_ref, hi_ref, hs2_ref, dv_ref, b2_ref, o_ref):
    s2 = lo_ref[0] + hi_ref[0] + hs2_ref[...]
    o_ref[...] = dv_ref[...] * s2 + b2_ref[...]


def _final_call(s2, hs2, dinv, b2):
    return pl.pallas_call(
        _final_body,
        grid=(N // BLK,),
        in_specs=[pl.BlockSpec((1, BLK, F_OUT), lambda i: (0, i, 0)),
                  pl.BlockSpec((1, BLK, F_OUT), lambda i: (1, i, 0)),
                  pl.BlockSpec((BLK, F_OUT), lambda i: (i, 0)),
                  pl.BlockSpec((BLK, 1), lambda i: (i, 0)),
                  pl.BlockSpec((1, F_OUT), lambda i: (0, 0))],
        out_specs=pl.BlockSpec((BLK, F_OUT), lambda i: (i, 0)),
        out_shape=jax.ShapeDtypeStruct((N, F_OUT), _f32),
    )(s2, s2, hs2, dinv, b2)


def kernel(x, edge_index, W1, b1, W2, b2):
    src = edge_index[0]
    dst = edge_index[1]
    src_t1 = src.reshape(NS, 125, 80)
    dst_t1 = dst.reshape(NS, 125, 80)
    src_t2 = src.reshape(2 * NS, 50, 100)
    dst_t2 = dst.reshape(2 * NS, 50, 100)

    partials = _deg_call(edge_index)
    dinv = _dinv_call(partials)
    hs1 = _mmscale_call(dinv, x, W1)
    s1 = _agg_call(hs1, src_t1, dst_t1, True)
    hs2 = _layer_call(s1, hs1, dinv, b1.reshape(1, HID), W2)
    s2 = _agg_call(hs2, src_t2, dst_t2, False)
    return _final_call(s2, hs2, dinv, b2.reshape(1, F_OUT))
